# baseline (speedup 1.0000x reference)
import jax
import jax.numpy as jnp
from jax import lax
from jax.experimental import pallas as pl
from jax.experimental.pallas import tpu as pltpu
from jax.experimental.pallas import tpu_sc as plsc

_D = 300
_PAIR = 2 * _D
_B, _T = 1024, 200
_N = _B * _T
_NC, _NS = 2, 16
_NW = _NC * _NS
_CHUNK = 40
_CPAD = 48
_TPB = _T // _CHUNK
_NCHUNK = _N // _CHUNK
_PER_W = _NCHUNK // _NW
_L = 16
_NPAIR = 400000 // 2


def _gather_body(pairs_hbm, idx_hbm, out_hbm,
                 idx_a, pidx_a, rows_a, obuf_a,
                 idx_b, pidx_b, rows_b, obuf_b,
                 sem_a, sem_b):
    c = lax.axis_index("c")
    s = lax.axis_index("s")
    wid = s * _NC + c

    def stage_and_start(u, idx_v, pidx_v, rows_p, sem):
        t = wid + _NW * u
        pltpu.sync_copy(idx_hbm.at[pl.ds(t * _CHUNK, _CHUNK)],
                        idx_v.at[pl.ds(0, _CHUNK)])
        for g in range(_CPAD // _L):
            vec = jax.lax.shift_right_logical(idx_v[pl.ds(g * _L, _L)], 1)
            pidx_v[pl.ds(g * _L, _L)] = jnp.clip(vec, 0, _NPAIR - 1)
        pltpu.async_copy(pairs_hbm.at[pidx_v], rows_p, sem)

    def finish(u, idx_v, pidx_v, rows_p, obuf, sem):
        pltpu.make_async_copy(pairs_hbm.at[pidx_v], rows_p, sem).wait()
        for g in range(_CPAD // _L):
            vec = idx_v[pl.ds(g * _L, _L)]
            nl = min(_L, _CHUNK - g * _L)
            for l in range(nl):
                j = g * _L + l
                off = (vec[l] & 1) * _D
                for k in range(0, _D, _L):
                    kk = min(k, _D - _L)
                    obuf[j, pl.ds(kk, _L)] = rows_p[j, pl.ds(off + kk, _L)]
        t = wid + _NW * u
        pltpu.sync_copy(
            obuf, out_hbm.at[t // _TPB, pl.ds((t % _TPB) * _CHUNK, _CHUNK)])

    stage_and_start(0, idx_a, pidx_a, rows_a, sem_a)

    def step(u2, carry):
        ua = 2 * u2
        ub = ua + 1
        stage_and_start(ub, idx_b, pidx_b, rows_b, sem_b)
        finish(ua, idx_a, pidx_a, rows_a, obuf_a, sem_a)

        @pl.when(ub + 1 < _PER_W)
        def _():
            stage_and_start(ub + 1, idx_a, pidx_a, rows_a, sem_a)

        finish(ub, idx_b, pidx_b, rows_b, obuf_b, sem_b)
        return carry

    lax.fori_loop(0, _PER_W // 2, step, 0)


@jax.jit
def kernel(x, table):
    idx = jnp.reshape(x, (_N,)).astype(jnp.int32)
    pairs = jnp.reshape(table, (_NPAIR, _PAIR))
    mesh = plsc.VectorSubcoreMesh(core_axis_name="c", subcore_axis_name="s")
    gather = pl.kernel(
        _gather_body,
        mesh=mesh,
        out_type=jax.ShapeDtypeStruct((_B, _T, _D), jnp.float32),
        scratch_types=[
            pltpu.VMEM((_CPAD,), jnp.int32),
            pltpu.VMEM((_CPAD,), jnp.int32),
            pltpu.VMEM((_CPAD, _PAIR), jnp.float32),
            pltpu.VMEM((_CHUNK, _D), jnp.float32),
            pltpu.VMEM((_CPAD,), jnp.int32),
            pltpu.VMEM((_CPAD,), jnp.int32),
            pltpu.VMEM((_CPAD, _PAIR), jnp.float32),
            pltpu.VMEM((_CHUNK, _D), jnp.float32),
            pltpu.SemaphoreType.DMA,
            pltpu.SemaphoreType.DMA,
        ],
        compiler_params=pltpu.CompilerParams(use_tc_tiling_on_sc=False),
    )
    return gather(pairs, idx)


# speedup vs baseline: 1.5673x; 1.5673x over previous
import jax
import jax.numpy as jnp
from jax import lax
from jax.experimental import pallas as pl
from jax.experimental.pallas import tpu as pltpu
from jax.experimental.pallas import tpu_sc as plsc

_D = 300
_PAIR = 2 * _D
_N = 1024 * 200
_NC, _NS = 2, 16
_NW = _NC * _NS
_CHUNK = 16
_NCHUNK = _N // _CHUNK
_PER_W = _NCHUNK // _NW
_L = 16


def _gather_body(pairs_hbm, idx_hbm, out_hbm,
                 idx_a, pidx_a, rows_a, obuf_a,
                 idx_b, pidx_b, rows_b, obuf_b,
                 sem_a, sem_b):
    c = lax.axis_index("c")
    s = lax.axis_index("s")
    wid = s * _NC + c

    def stage_and_start(u, idx_v, pidx_v, rows_p, sem):
        base = (wid + _NW * u) * _CHUNK
        pltpu.sync_copy(idx_hbm.at[pl.ds(base, _CHUNK)], idx_v)
        for g in range(_CHUNK // _L):
            pidx_v[pl.ds(g * _L, _L)] = jax.lax.shift_right_logical(
                idx_v[pl.ds(g * _L, _L)], 1)
        pltpu.async_copy(pairs_hbm.at[pidx_v], rows_p, sem)

    def finish(u, idx_v, pidx_v, rows_p, obuf, sem):
        pltpu.make_async_copy(pairs_hbm.at[pidx_v], rows_p, sem).wait()
        for g in range(_CHUNK // _L):
            vec = idx_v[pl.ds(g * _L, _L)]
            for l in range(_L):
                j = g * _L + l
                p = jnp.full((_L,), jax.lax.convert_element_type(
                    vec[l] & 1, jnp.float32), jnp.float32)
                for k in range(0, _D, _L):
                    kk = min(k, _D - _L)
                    a = rows_p[j, pl.ds(kk, _L)]
                    b = rows_p[j, pl.ds(_D + kk, _L)]
                    obuf[j, pl.ds(kk, _L)] = a + p * (b - a)
        base = (wid + _NW * u) * _CHUNK
        pltpu.sync_copy(obuf, out_hbm.at[pl.ds(base, _CHUNK)])

    stage_and_start(0, idx_a, pidx_a, rows_a, sem_a)

    def step(u2, carry):
        ua = 2 * u2
        ub = ua + 1
        stage_and_start(ub, idx_b, pidx_b, rows_b, sem_b)
        finish(ua, idx_a, pidx_a, rows_a, obuf_a, sem_a)

        @pl.when(ub + 1 < _PER_W)
        def _():
            stage_and_start(ub + 1, idx_a, pidx_a, rows_a, sem_a)

        finish(ub, idx_b, pidx_b, rows_b, obuf_b, sem_b)
        return carry

    lax.fori_loop(0, _PER_W // 2, step, 0)


@jax.jit
def kernel(x, table):
    idx = jnp.reshape(x, (_N,)).astype(jnp.int32)
    pairs = jnp.reshape(table, (table.shape[0] // 2, _PAIR))
    mesh = plsc.VectorSubcoreMesh(core_axis_name="c", subcore_axis_name="s")
    gather = pl.kernel(
        _gather_body,
        mesh=mesh,
        out_type=jax.ShapeDtypeStruct((_N, _D), jnp.float32),
        scratch_types=[
            pltpu.VMEM((_CHUNK,), jnp.int32),
            pltpu.VMEM((_CHUNK,), jnp.int32),
            pltpu.VMEM((_CHUNK, _PAIR), jnp.float32),
            pltpu.VMEM((_CHUNK, _D), jnp.float32),
            pltpu.VMEM((_CHUNK,), jnp.int32),
            pltpu.VMEM((_CHUNK,), jnp.int32),
            pltpu.VMEM((_CHUNK, _PAIR), jnp.float32),
            pltpu.VMEM((_CHUNK, _D), jnp.float32),
            pltpu.SemaphoreType.DMA,
            pltpu.SemaphoreType.DMA,
        ],
        compiler_params=pltpu.CompilerParams(use_tc_tiling_on_sc=False),
    )
    out = gather(pairs, idx)
    return out.reshape(x.shape + (_D,))


# speedup vs baseline: 1.5816x; 1.0092x over previous
import jax
import jax.numpy as jnp
from jax import lax
from jax.experimental import pallas as pl
from jax.experimental.pallas import tpu as pltpu
from jax.experimental.pallas import tpu_sc as plsc

_D = 300
_PAIR = 2 * _D
_N = 1024 * 200
_NC, _NS = 2, 16
_NW = _NC * _NS
_CHUNK = 32
_NCHUNK = _N // _CHUNK
_PER_W = _NCHUNK // _NW
_L = 16


def _gather_body(pairs_hbm, idx_hbm, out_hbm,
                 idx_a, pidx_a, rows_a, obuf_a,
                 idx_b, pidx_b, rows_b, obuf_b,
                 sem_a, sem_b):
    c = lax.axis_index("c")
    s = lax.axis_index("s")
    wid = s * _NC + c

    def stage_and_start(u, idx_v, pidx_v, rows_p, sem):
        base = (wid + _NW * u) * _CHUNK
        pltpu.sync_copy(idx_hbm.at[pl.ds(base, _CHUNK)], idx_v)
        for g in range(_CHUNK // _L):
            pidx_v[pl.ds(g * _L, _L)] = jax.lax.shift_right_logical(
                idx_v[pl.ds(g * _L, _L)], 1)
        pltpu.async_copy(pairs_hbm.at[pidx_v], rows_p, sem)

    def finish(u, idx_v, pidx_v, rows_p, obuf, sem):
        pltpu.make_async_copy(pairs_hbm.at[pidx_v], rows_p, sem).wait()
        for g in range(_CHUNK // _L):
            vec = idx_v[pl.ds(g * _L, _L)]
            for l in range(_L):
                j = g * _L + l
                p = jnp.full((_L,), jax.lax.convert_element_type(
                    vec[l] & 1, jnp.float32), jnp.float32)
                for k in range(0, _D, _L):
                    kk = min(k, _D - _L)
                    a = rows_p[j, pl.ds(kk, _L)]
                    b = rows_p[j, pl.ds(_D + kk, _L)]
                    obuf[j, pl.ds(kk, _L)] = a + p * (b - a)
        base = (wid + _NW * u) * _CHUNK
        pltpu.sync_copy(obuf, out_hbm.at[pl.ds(base, _CHUNK)])

    stage_and_start(0, idx_a, pidx_a, rows_a, sem_a)

    def step(u2, carry):
        ua = 2 * u2
        ub = ua + 1
        stage_and_start(ub, idx_b, pidx_b, rows_b, sem_b)
        finish(ua, idx_a, pidx_a, rows_a, obuf_a, sem_a)

        @pl.when(ub + 1 < _PER_W)
        def _():
            stage_and_start(ub + 1, idx_a, pidx_a, rows_a, sem_a)

        finish(ub, idx_b, pidx_b, rows_b, obuf_b, sem_b)
        return carry

    lax.fori_loop(0, _PER_W // 2, step, 0)


@jax.jit
def kernel(x, table):
    idx = jnp.reshape(x, (_N,)).astype(jnp.int32)
    pairs = jnp.reshape(table, (table.shape[0] // 2, _PAIR))
    mesh = plsc.VectorSubcoreMesh(core_axis_name="c", subcore_axis_name="s")
    gather = pl.kernel(
        _gather_body,
        mesh=mesh,
        out_type=jax.ShapeDtypeStruct((_N, _D), jnp.float32),
        scratch_types=[
            pltpu.VMEM((_CHUNK,), jnp.int32),
            pltpu.VMEM((_CHUNK,), jnp.int32),
            pltpu.VMEM((_CHUNK, _PAIR), jnp.float32),
            pltpu.VMEM((_CHUNK, _D), jnp.float32),
            pltpu.VMEM((_CHUNK,), jnp.int32),
            pltpu.VMEM((_CHUNK,), jnp.int32),
            pltpu.VMEM((_CHUNK, _PAIR), jnp.float32),
            pltpu.VMEM((_CHUNK, _D), jnp.float32),
            pltpu.SemaphoreType.DMA,
            pltpu.SemaphoreType.DMA,
        ],
        compiler_params=pltpu.CompilerParams(use_tc_tiling_on_sc=False),
    )
    out = gather(pairs, idx)
    return out.reshape(x.shape + (_D,))
